# baseline (speedup 1.0000x reference)
import functools

import jax
import jax.numpy as jnp
from jax import lax
from jax.experimental import pallas as pl
from jax.experimental.pallas import tpu as pltpu
from jax.experimental.pallas import tpu_sc as plsc

TOK = 4096
HS = 1024
E = 8
M = 512
D = 256
TPE = TOK // E
EH = E // 2
HTOK = TOK // 2

NC = 2
NS = 16
NW = NC * NS
RPW = HTOK // NW
WPE = TPE // RPW


@functools.lru_cache(maxsize=None)
def _make_sc_gather(h):
    mesh = plsc.VectorSubcoreMesh(core_axis_name="c", subcore_axis_name="s")

    @functools.partial(
        pl.kernel,
        out_type=jax.ShapeDtypeStruct((HTOK, HS), jnp.float32),
        mesh=mesh,
        scratch_types=[
            pltpu.VMEM((RPW,), jnp.int32),
            pltpu.VMEM((RPW // 2, HS), jnp.float32),
            pltpu.VMEM((RPW // 2, HS), jnp.float32),
            pltpu.SemaphoreType.DMA,
            pltpu.SemaphoreType.DMA,
        ],
    )
    def _sc_gather(ctx_hbm, eidx_hbm, xs_hbm,
                   idx_v, buf_a, buf_b, sem_a, sem_b):
        wid = lax.axis_index("s") * NC + lax.axis_index("c")
        e = h * EH + wid // WPE
        col = (wid % WPE) * RPW
        base = wid * RPW
        half = RPW // 2
        pltpu.sync_copy(eidx_hbm.at[e, pl.ds(col, RPW)], idx_v)
        cp_a = pltpu.async_copy(ctx_hbm.at[idx_v.at[pl.ds(0, half)]],
                                buf_a, sem_a)
        cp_b = pltpu.async_copy(ctx_hbm.at[idx_v.at[pl.ds(half, half)]],
                                buf_b, sem_b)
        cp_a.wait()
        pltpu.sync_copy(buf_a, xs_hbm.at[pl.ds(base, half)])
        cp_b.wait()
        pltpu.sync_copy(buf_b, xs_hbm.at[pl.ds(base + half, half)])

    return _sc_gather


@functools.lru_cache(maxsize=None)
def _make_sc_scatter():
    mesh = plsc.VectorSubcoreMesh(core_axis_name="c", subcore_axis_name="s")
    NCH = D // 16

    @functools.partial(
        pl.kernel,
        out_type=jax.ShapeDtypeStruct((TOK, 2 * D), jnp.float32),
        mesh=mesh,
        scratch_types=[
            pltpu.VMEM((2, RPW), jnp.int32),
            pltpu.VMEM((2, RPW, 16), jnp.float32),
            pltpu.VMEM((RPW, 2 * D), jnp.float32),
            pltpu.VMEM((RPW, 2 * D), jnp.float32),
            pltpu.VMEM((RPW, D), jnp.float32),
            pltpu.VMEM((RPW, D), jnp.float32),
            pltpu.VMEM((1, 16), jnp.float32),
            pltpu.VMEM((1, 16), jnp.float32),
            pltpu.SemaphoreType.DMA,
            pltpu.SemaphoreType.DMA,
            pltpu.SemaphoreType.DMA,
            pltpu.SemaphoreType.DMA,
            pltpu.SemaphoreType.DMA,
        ],
    )
    def _sc_scatter(b1_hbm, b2_hbm, c01_hbm, c02_hbm, eidx_hbm,
                    ent1_hbm, ent2_hbm, psr_hbm, atk_hbm, out_hbm,
                    idx_v, c0_v, buf_a, buf_b, ori_p, ori_a,
                    ent1_v, ent2_v,
                    sem_a, sem_b, sem_l, sem_p, sem_k):
        wid = lax.axis_index("s") * NC + lax.axis_index("c")
        eh = wid // WPE
        col = (wid % WPE) * RPW
        base = wid * RPW
        pltpu.sync_copy(ent1_hbm, ent1_v)
        pltpu.sync_copy(ent2_hbm, ent2_v)
        ent = ent1_v[0] + ent2_v[0]
        pltpu.sync_copy(eidx_hbm.at[eh, pl.ds(col, RPW)],
                        idx_v.at[0])
        pltpu.sync_copy(eidx_hbm.at[EH + eh, pl.ds(col, RPW)],
                        idx_v.at[1])
        pltpu.sync_copy(c01_hbm.at[pl.ds(base, RPW)], c0_v.at[0])
        pltpu.sync_copy(c02_hbm.at[pl.ds(base, RPW)], c0_v.at[1])

        def _blend(buf, hi):
            def _row(r, _):
                cb = c0_v[hi, r]
                for j in range(NCH):
                    sl = pl.ds(j * 16, 16)
                    buf[r, sl] = (ori_p[r, sl] * cb
                                  + (buf[r, sl] + ent))
                for j in range(NCH):
                    sl = pl.ds(j * 16, 16)
                    so = pl.ds(D + j * 16, 16)
                    buf[r, so] = (ori_a[r, sl] * cb
                                  + (buf[r, so] + ent))
                return 0
            lax.fori_loop(0, RPW, _row, 0)

        cp_la = pltpu.async_copy(b1_hbm.at[pl.ds(base, RPW)], buf_a, sem_l)
        cp_p = pltpu.async_copy(psr_hbm.at[idx_v.at[0]], ori_p, sem_p)
        cp_k = pltpu.async_copy(atk_hbm.at[idx_v.at[0]], ori_a, sem_k)
        cp_la.wait()
        cp_p.wait()
        cp_k.wait()
        _blend(buf_a, 0)
        cp_a = pltpu.async_copy(buf_a, out_hbm.at[idx_v.at[0]], sem_a)
        cp_lb = pltpu.async_copy(b2_hbm.at[pl.ds(base, RPW)], buf_b, sem_l)
        cp_p = pltpu.async_copy(psr_hbm.at[idx_v.at[1]], ori_p, sem_p)
        cp_k = pltpu.async_copy(atk_hbm.at[idx_v.at[1]], ori_a, sem_k)
        cp_lb.wait()
        cp_p.wait()
        cp_k.wait()
        _blend(buf_b, 1)
        cp_b = pltpu.async_copy(buf_b, out_hbm.at[idx_v.at[1]], sem_b)
        cp_a.wait()
        cp_b.wait()

    return _sc_scatter


def _make_expert_body(h):
    def _expert_body(x_ref, dW_ref, db_ref, plut_ref,
                     alut_ref, w1_ref, b1_ref, w2_ref, b2_ref,
                     out_ref, c0_ref, ent_ref, acc_ref):
        e = pl.program_id(0)
        xb = x_ref[...].astype(jnp.bfloat16)
        logits = jnp.dot(xb, dW_ref[0].astype(jnp.bfloat16),
                         preferred_element_type=jnp.float32) + db_ref[0]
        m = jnp.max(logits, axis=-1, keepdims=True)
        z = logits - m
        ez = jnp.exp(z)
        s = jnp.sum(ez, axis=-1, keepdims=True)
        spt = ez / s
        pspt = z - jnp.log(s)
        ent_blk = jnp.sum(-pspt * spt) * (1.0 / (TPE * M))

        @pl.when(e == 0)
        def _():
            acc_ref[0] = 0.0

        acc_ref[0] += ent_blk

        hh = jnp.maximum(jnp.dot(xb, w1_ref[...].astype(jnp.bfloat16),
                                 preferred_element_type=jnp.float32)
                         + b1_ref[...], 0.0)
        u = jnp.dot(hh.astype(jnp.bfloat16), w2_ref[...].astype(jnp.bfloat16),
                    preferred_element_type=jnp.float32) + b2_ref[...]
        um = jnp.max(u, axis=-1, keepdims=True)
        ue = jnp.exp(u - um)
        c = ue / jnp.sum(ue, axis=-1, keepdims=True)
        c0 = c[:, 0:1]
        c1 = c[:, 1:2]

        sptb = (spt * c1).astype(jnp.bfloat16)
        out_ref[:, :D] = jnp.dot(sptb, plut_ref[0].astype(jnp.bfloat16),
                                 preferred_element_type=jnp.float32)
        out_ref[:, D:] = jnp.dot(sptb, alut_ref[0].astype(jnp.bfloat16),
                                 preferred_element_type=jnp.float32)
        c0_ref[...] = jnp.broadcast_to(c0, (TPE, 16))

        @pl.when(e == EH - 1)
        def _():
            ent_ref[...] = jnp.full((1, 16), acc_ref[0], jnp.float32)

    return _expert_body


@functools.lru_cache(maxsize=None)
def _make_tc_experts(h):
    idx0 = lambda e: (0, 0)
    idxe = lambda e: (e, 0)
    idxe3 = lambda e: (h * EH + e, 0, 0)
    body = _make_expert_body(h)

    def _call(xs, dec_W, dec_b, psr_lut, atk_lut, w1, b1, w2, b2):
        return pl.pallas_call(
            body,
            grid=(EH,),
            in_specs=[
                pl.BlockSpec((TPE, HS), idxe),
                pl.BlockSpec((1, HS, M), idxe3),
                pl.BlockSpec((1, 1, M), idxe3),
                pl.BlockSpec((1, M, D), idxe3),
                pl.BlockSpec((1, M, D), idxe3),
                pl.BlockSpec((HS, 64), idx0),
                pl.BlockSpec((1, 64), idx0),
                pl.BlockSpec((64, 2), idx0),
                pl.BlockSpec((1, 2), idx0),
            ],
            out_specs=[
                pl.BlockSpec((TPE, 2 * D), idxe),
                pl.BlockSpec((TPE, 16), idxe),
                pl.BlockSpec((1, 16), idx0),
            ],
            out_shape=[
                jax.ShapeDtypeStruct((HTOK, 2 * D), jnp.float32),
                jax.ShapeDtypeStruct((HTOK, 16), jnp.float32),
                jax.ShapeDtypeStruct((1, 16), jnp.float32),
            ],
            scratch_shapes=[pltpu.SMEM((1,), jnp.float32)],
        )(xs, dec_W, dec_b, psr_lut, atk_lut, w1, b1, w2, b2)

    return _call


def kernel(ctx, expert_idx, dec_W, dec_b, copy_W1, copy_b1, copy_W2, copy_b2,
           psr_lut, atk_lut, ori_psr, ori_atk):
    eidx = expert_idx.astype(jnp.int32)
    db3 = dec_b.reshape(E, 1, M)
    b1r = copy_b1.reshape(1, 64)
    b2r = copy_b2.reshape(1, 2)
    xs1 = _make_sc_gather(0)(ctx, eidx)
    xs2 = _make_sc_gather(1)(ctx, eidx)
    b1, c01, ent1 = _make_tc_experts(0)(xs1, dec_W, db3, psr_lut, atk_lut,
                                        copy_W1, b1r, copy_W2, b2r)
    b2, c02, ent2 = _make_tc_experts(1)(xs2, dec_W, db3, psr_lut, atk_lut,
                                        copy_W1, b1r, copy_W2, b2r)
    return _make_sc_scatter()(b1, b2, c01, c02, eidx, ent1, ent2,
                              ori_psr, ori_atk)


# speedup vs baseline: 1.0080x; 1.0080x over previous
import functools

import jax
import jax.numpy as jnp
from jax import lax
from jax.experimental import pallas as pl
from jax.experimental.pallas import tpu as pltpu
from jax.experimental.pallas import tpu_sc as plsc

TOK = 4096
HS = 1024
E = 8
M = 512
D = 256
TPE = TOK // E
EH = E // 2
HTOK = TOK // 2

NC = 2
NS = 16
NW = NC * NS
RPW = HTOK // NW
WPE = TPE // RPW


@functools.lru_cache(maxsize=None)
def _make_sc_gather(h):
    mesh = plsc.VectorSubcoreMesh(core_axis_name="c", subcore_axis_name="s")

    @functools.partial(
        pl.kernel,
        out_type=[
            jax.ShapeDtypeStruct((HTOK, HS), jnp.float32),
            jax.ShapeDtypeStruct((HTOK, D), jnp.float32),
            jax.ShapeDtypeStruct((HTOK, D), jnp.float32),
        ],
        mesh=mesh,
        scratch_types=[
            pltpu.VMEM((RPW,), jnp.int32),
            pltpu.VMEM((RPW // 2, HS), jnp.float32),
            pltpu.VMEM((RPW // 2, HS), jnp.float32),
            pltpu.VMEM((RPW, D), jnp.float32),
            pltpu.VMEM((RPW, D), jnp.float32),
            pltpu.SemaphoreType.DMA,
            pltpu.SemaphoreType.DMA,
            pltpu.SemaphoreType.DMA,
            pltpu.SemaphoreType.DMA,
        ],
    )
    def _sc_gather(ctx_hbm, psr_hbm, atk_hbm, eidx_hbm,
                   xs_hbm, ps_hbm, as_hbm,
                   idx_v, buf_a, buf_b, emb_p, emb_a,
                   sem_a, sem_b, sem_p, sem_k):
        wid = lax.axis_index("s") * NC + lax.axis_index("c")
        e = h * EH + wid // WPE
        col = (wid % WPE) * RPW
        base = wid * RPW
        half = RPW // 2
        pltpu.sync_copy(eidx_hbm.at[e, pl.ds(col, RPW)], idx_v)
        cp_p = pltpu.async_copy(psr_hbm.at[idx_v], emb_p, sem_p)
        cp_k = pltpu.async_copy(atk_hbm.at[idx_v], emb_a, sem_k)
        cp_a = pltpu.async_copy(ctx_hbm.at[idx_v.at[pl.ds(0, half)]],
                                buf_a, sem_a)
        cp_b = pltpu.async_copy(ctx_hbm.at[idx_v.at[pl.ds(half, half)]],
                                buf_b, sem_b)
        cp_a.wait()
        pltpu.sync_copy(buf_a, xs_hbm.at[pl.ds(base, half)])
        cp_b.wait()
        pltpu.sync_copy(buf_b, xs_hbm.at[pl.ds(base + half, half)])
        cp_p.wait()
        pltpu.sync_copy(emb_p, ps_hbm.at[pl.ds(base, RPW)])
        cp_k.wait()
        pltpu.sync_copy(emb_a, as_hbm.at[pl.ds(base, RPW)])

    return _sc_gather


@functools.lru_cache(maxsize=None)
def _make_sc_scatter():
    mesh = plsc.VectorSubcoreMesh(core_axis_name="c", subcore_axis_name="s")

    @functools.partial(
        pl.kernel,
        out_type=jax.ShapeDtypeStruct((TOK, 2 * D), jnp.float32),
        mesh=mesh,
        scratch_types=[
            pltpu.VMEM((RPW,), jnp.int32),
            pltpu.VMEM((RPW,), jnp.int32),
            pltpu.VMEM((RPW, 2 * D), jnp.float32),
            pltpu.VMEM((RPW, 2 * D), jnp.float32),
            pltpu.VMEM((1, 16), jnp.float32),
            pltpu.VMEM((1, 16), jnp.float32),
            pltpu.SemaphoreType.DMA,
            pltpu.SemaphoreType.DMA,
            pltpu.SemaphoreType.DMA,
            pltpu.SemaphoreType.DMA,
        ],
    )
    def _sc_scatter(b1_hbm, b2_hbm, eidx_hbm, ent1_hbm, ent2_hbm, out_hbm,
                    idx_a, idx_b, buf_a, buf_b, ent1_v, ent2_v,
                    sem_a, sem_b, sem_la, sem_lb):
        wid = lax.axis_index("s") * NC + lax.axis_index("c")
        eh = wid // WPE
        col = (wid % WPE) * RPW
        base = wid * RPW
        pltpu.sync_copy(ent1_hbm, ent1_v)
        pltpu.sync_copy(ent2_hbm, ent2_v)
        ent = ent1_v[0] + ent2_v[0]
        pltpu.sync_copy(eidx_hbm.at[eh, pl.ds(col, RPW)], idx_a)
        pltpu.sync_copy(eidx_hbm.at[EH + eh, pl.ds(col, RPW)], idx_b)
        cp_la = pltpu.async_copy(b1_hbm.at[pl.ds(base, RPW)], buf_a, sem_la)
        cp_lb = pltpu.async_copy(b2_hbm.at[pl.ds(base, RPW)], buf_b, sem_lb)

        def _add_ent(buf):
            def _row(r, _):
                for j in range(2 * D // 16):
                    buf[r, pl.ds(j * 16, 16)] += ent
                return 0
            lax.fori_loop(0, RPW, _row, 0)

        cp_la.wait()
        _add_ent(buf_a)
        cp_a = pltpu.async_copy(buf_a, out_hbm.at[idx_a], sem_a)
        cp_lb.wait()
        _add_ent(buf_b)
        cp_b = pltpu.async_copy(buf_b, out_hbm.at[idx_b], sem_b)
        cp_a.wait()
        cp_b.wait()

    return _sc_scatter


TS = 4
TB = TPE // TS


def _make_expert_body(h):
    def _expert_body(x_ref, opsr_ref, oatk_ref, dW_ref, db_ref, plut_ref,
                     alut_ref, w1_ref, b1_ref, w2_ref, b2_ref,
                     out_ref, ent_ref, acc_ref):
        e = pl.program_id(0)
        t = pl.program_id(1)
        xb = x_ref[...].astype(jnp.bfloat16)
        logits = jnp.dot(xb, dW_ref[0].astype(jnp.bfloat16),
                         preferred_element_type=jnp.float32) + db_ref[0]
        m = jnp.max(logits, axis=-1, keepdims=True)
        z = logits - m
        ez = jnp.exp(z)
        s = jnp.sum(ez, axis=-1, keepdims=True)
        spt = ez / s
        pspt = z - jnp.log(s)
        ent_blk = jnp.sum(-pspt * spt) * (1.0 / (TPE * M))

        @pl.when((e == 0) & (t == 0))
        def _():
            acc_ref[0] = 0.0

        acc_ref[0] += ent_blk

        sptb = spt.astype(jnp.bfloat16)
        psr = jnp.dot(sptb, plut_ref[0].astype(jnp.bfloat16),
                      preferred_element_type=jnp.float32)
        atk = jnp.dot(sptb, alut_ref[0].astype(jnp.bfloat16),
                      preferred_element_type=jnp.float32)

        hh = jnp.maximum(jnp.dot(xb, w1_ref[...].astype(jnp.bfloat16),
                                 preferred_element_type=jnp.float32)
                         + b1_ref[...], 0.0)
        u = jnp.dot(hh.astype(jnp.bfloat16), w2_ref[...].astype(jnp.bfloat16),
                    preferred_element_type=jnp.float32) + b2_ref[...]
        um = jnp.max(u, axis=-1, keepdims=True)
        ue = jnp.exp(u - um)
        c = ue / jnp.sum(ue, axis=-1, keepdims=True)
        c0 = c[:, 0:1]
        c1 = c[:, 1:2]
        out_ref[:, :D] = opsr_ref[...] * c0 + psr * c1
        out_ref[:, D:] = oatk_ref[...] * c0 + atk * c1

        @pl.when((e == EH - 1) & (t == TS - 1))
        def _():
            ent_ref[...] = jnp.full((1, 16), acc_ref[0], jnp.float32)

    return _expert_body


@functools.lru_cache(maxsize=None)
def _make_tc_experts(h):
    idx0 = lambda e, t: (0, 0)
    idxe = lambda e, t: (e * TS + t, 0)
    idxe3 = lambda e, t: (h * EH + e, 0, 0)
    body = _make_expert_body(h)

    def _call(xs, ops, oat, dec_W, dec_b, psr_lut, atk_lut, w1, b1, w2, b2):
        return pl.pallas_call(
            body,
            grid=(EH, TS),
            in_specs=[
                pl.BlockSpec((TB, HS), idxe),
                pl.BlockSpec((TB, D), idxe),
                pl.BlockSpec((TB, D), idxe),
                pl.BlockSpec((1, HS, M), idxe3),
                pl.BlockSpec((1, 1, M), idxe3),
                pl.BlockSpec((1, M, D), idxe3),
                pl.BlockSpec((1, M, D), idxe3),
                pl.BlockSpec((HS, 64), idx0),
                pl.BlockSpec((1, 64), idx0),
                pl.BlockSpec((64, 2), idx0),
                pl.BlockSpec((1, 2), idx0),
            ],
            out_specs=[
                pl.BlockSpec((TB, 2 * D), idxe),
                pl.BlockSpec((1, 16), idx0),
            ],
            out_shape=[
                jax.ShapeDtypeStruct((HTOK, 2 * D), jnp.float32),
                jax.ShapeDtypeStruct((1, 16), jnp.float32),
            ],
            scratch_shapes=[pltpu.SMEM((1,), jnp.float32)],
        )(xs, ops, oat, dec_W, dec_b, psr_lut, atk_lut, w1, b1, w2, b2)

    return _call


def kernel(ctx, expert_idx, dec_W, dec_b, copy_W1, copy_b1, copy_W2, copy_b2,
           psr_lut, atk_lut, ori_psr, ori_atk):
    eidx = expert_idx.astype(jnp.int32)
    db3 = dec_b.reshape(E, 1, M)
    b1r = copy_b1.reshape(1, 64)
    b2r = copy_b2.reshape(1, 2)
    xs1, ops1, oat1 = _make_sc_gather(0)(ctx, ori_psr, ori_atk, eidx)
    xs2, ops2, oat2 = _make_sc_gather(1)(ctx, ori_psr, ori_atk, eidx)
    b1, ent1 = _make_tc_experts(0)(xs1, ops1, oat1, dec_W, db3, psr_lut,
                                   atk_lut, copy_W1, b1r, copy_W2, b2r)
    b2, ent2 = _make_tc_experts(1)(xs2, ops2, oat2, dec_W, db3, psr_lut,
                                   atk_lut, copy_W1, b1r, copy_W2, b2r)
    return _make_sc_scatter()(b1, b2, eidx, ent1, ent2)


# speedup vs baseline: 1.2940x; 1.2837x over previous
import functools

import jax
import jax.numpy as jnp
from jax import lax
from jax.experimental import pallas as pl
from jax.experimental.pallas import tpu as pltpu
from jax.experimental.pallas import tpu_sc as plsc

TOK = 4096
HS = 1024
E = 8
M = 512
D = 256
TPE = TOK // E
EH = E // 2
HTOK = TOK // 2

NC = 2
NS = 16
NW = NC * NS
RPW = HTOK // NW
WPE = TPE // RPW


@functools.lru_cache(maxsize=None)
def _make_sc_gather():
    mesh = plsc.VectorSubcoreMesh(core_axis_name="c", subcore_axis_name="s")

    @functools.partial(
        pl.kernel,
        out_type=[
            jax.ShapeDtypeStruct((HTOK, HS), jnp.float32),
            jax.ShapeDtypeStruct((HTOK, D), jnp.float32),
            jax.ShapeDtypeStruct((HTOK, D), jnp.float32),
        ],
        mesh=mesh,
        scratch_types=[
            pltpu.VMEM((RPW,), jnp.int32),
            pltpu.VMEM((RPW // 2, HS), jnp.float32),
            pltpu.VMEM((RPW // 2, HS), jnp.float32),
            pltpu.VMEM((RPW, D), jnp.float32),
            pltpu.VMEM((RPW, D), jnp.float32),
            pltpu.SemaphoreType.DMA,
            pltpu.SemaphoreType.DMA,
            pltpu.SemaphoreType.DMA,
            pltpu.SemaphoreType.DMA,
        ],
    )
    def _sc_gather(ctx_hbm, psr_hbm, atk_hbm, eidx_hbm,
                   xs_hbm, ps_hbm, as_hbm,
                   idx_v, buf_a, buf_b, emb_p, emb_a,
                   sem_a, sem_b, sem_p, sem_k):
        wid = lax.axis_index("s") * NC + lax.axis_index("c")
        e = wid // WPE
        col = (wid % WPE) * RPW
        base = wid * RPW
        half = RPW // 2
        pltpu.sync_copy(eidx_hbm.at[e, pl.ds(col, RPW)], idx_v)
        cp_p = pltpu.async_copy(psr_hbm.at[idx_v], emb_p, sem_p)
        cp_k = pltpu.async_copy(atk_hbm.at[idx_v], emb_a, sem_k)
        cp_a = pltpu.async_copy(ctx_hbm.at[idx_v.at[pl.ds(0, half)]],
                                buf_a, sem_a)
        cp_b = pltpu.async_copy(ctx_hbm.at[idx_v.at[pl.ds(half, half)]],
                                buf_b, sem_b)
        cp_a.wait()
        pltpu.sync_copy(buf_a, xs_hbm.at[pl.ds(base, half)])
        cp_b.wait()
        pltpu.sync_copy(buf_b, xs_hbm.at[pl.ds(base + half, half)])
        cp_p.wait()
        pltpu.sync_copy(emb_p, ps_hbm.at[pl.ds(base, RPW)])
        cp_k.wait()
        pltpu.sync_copy(emb_a, as_hbm.at[pl.ds(base, RPW)])

    return _sc_gather


@functools.lru_cache(maxsize=None)
def _make_sc_scatter():
    mesh = plsc.VectorSubcoreMesh(core_axis_name="c", subcore_axis_name="s")

    @functools.partial(
        pl.kernel,
        out_type=jax.ShapeDtypeStruct((TOK, 2 * D), jnp.float32),
        mesh=mesh,
        scratch_types=[
            pltpu.VMEM((RPW,), jnp.int32),
            pltpu.VMEM((RPW,), jnp.int32),
            pltpu.VMEM((RPW, 2 * D), jnp.float32),
            pltpu.VMEM((RPW, 2 * D), jnp.float32),
            pltpu.VMEM((1, 16), jnp.float32),
            pltpu.VMEM((1, 16), jnp.float32),
            pltpu.SemaphoreType.DMA,
            pltpu.SemaphoreType.DMA,
            pltpu.SemaphoreType.DMA,
            pltpu.SemaphoreType.DMA,
        ],
    )
    def _sc_scatter(b1_hbm, b2_hbm, eidx_hbm, ent1_hbm, ent2_hbm, out_hbm,
                    idx_a, idx_b, buf_a, buf_b, ent1_v, ent2_v,
                    sem_a, sem_b, sem_la, sem_lb):
        wid = lax.axis_index("s") * NC + lax.axis_index("c")
        eh = wid // WPE
        col = (wid % WPE) * RPW
        base = wid * RPW
        pltpu.sync_copy(ent1_hbm, ent1_v)
        pltpu.sync_copy(ent2_hbm, ent2_v)
        ent = ent1_v[0] + ent2_v[0]
        pltpu.sync_copy(eidx_hbm.at[eh, pl.ds(col, RPW)], idx_a)
        pltpu.sync_copy(eidx_hbm.at[EH + eh, pl.ds(col, RPW)], idx_b)
        cp_la = pltpu.async_copy(b1_hbm.at[pl.ds(base, RPW)], buf_a, sem_la)
        cp_lb = pltpu.async_copy(b2_hbm.at[pl.ds(base, RPW)], buf_b, sem_lb)

        def _add_ent(buf):
            def _row(r, _):
                for j in range(2 * D // 16):
                    buf[r, pl.ds(j * 16, 16)] += ent
                return 0
            lax.fori_loop(0, RPW, _row, 0)

        cp_la.wait()
        _add_ent(buf_a)
        cp_a = pltpu.async_copy(buf_a, out_hbm.at[idx_a], sem_a)
        cp_lb.wait()
        _add_ent(buf_b)
        cp_b = pltpu.async_copy(buf_b, out_hbm.at[idx_b], sem_b)
        cp_a.wait()
        cp_b.wait()

    return _sc_scatter


def _make_expert_body(h):
    def _expert_body(x_ref, opsr_ref, oatk_ref, dW_ref, db_ref, plut_ref,
                     alut_ref, w1_ref, b1_ref, w2_ref, b2_ref,
                     out_ref, ent_ref, acc_ref):
        e = pl.program_id(0)
        xb = x_ref[...].astype(jnp.bfloat16)
        logits = jnp.dot(xb, dW_ref[0].astype(jnp.bfloat16),
                         preferred_element_type=jnp.float32) + db_ref[0]
        m = jnp.max(logits, axis=-1, keepdims=True)
        z = logits - m
        ez = jnp.exp(z)
        s = jnp.sum(ez, axis=-1, keepdims=True)
        spt = ez / s
        pspt = z - jnp.log(s)
        ent_blk = jnp.sum(-pspt * spt) * (1.0 / (TPE * M))

        @pl.when(e == 0)
        def _():
            acc_ref[0] = 0.0

        acc_ref[0] += ent_blk

        sptb = spt.astype(jnp.bfloat16)
        psr = jnp.dot(sptb, plut_ref[0].astype(jnp.bfloat16),
                      preferred_element_type=jnp.float32)
        atk = jnp.dot(sptb, alut_ref[0].astype(jnp.bfloat16),
                      preferred_element_type=jnp.float32)

        hh = jnp.maximum(jnp.dot(xb, w1_ref[...].astype(jnp.bfloat16),
                                 preferred_element_type=jnp.float32)
                         + b1_ref[...], 0.0)
        u = jnp.dot(hh.astype(jnp.bfloat16), w2_ref[...].astype(jnp.bfloat16),
                    preferred_element_type=jnp.float32) + b2_ref[...]
        um = jnp.max(u, axis=-1, keepdims=True)
        ue = jnp.exp(u - um)
        c = ue / jnp.sum(ue, axis=-1, keepdims=True)
        c0 = c[:, 0:1]
        c1 = c[:, 1:2]
        out_ref[:, :D] = opsr_ref[...] * c0 + psr * c1
        out_ref[:, D:] = oatk_ref[...] * c0 + atk * c1

        @pl.when(e == EH - 1)
        def _():
            ent_ref[...] = jnp.full((1, 16), acc_ref[0], jnp.float32)

    return _expert_body


@functools.lru_cache(maxsize=None)
def _make_tc_experts(h):
    idx0 = lambda e: (0, 0)
    idxe = lambda e: (e, 0)
    idxe3 = lambda e: (h * EH + e, 0, 0)
    body = _make_expert_body(h)

    def _call(xs, ops, oat, dec_W, dec_b, psr_lut, atk_lut, w1, b1, w2, b2):
        return pl.pallas_call(
            body,
            grid=(EH,),
            in_specs=[
                pl.BlockSpec((TPE, HS), idxe),
                pl.BlockSpec((TPE, D), idxe),
                pl.BlockSpec((TPE, D), idxe),
                pl.BlockSpec((1, HS, M), idxe3),
                pl.BlockSpec((1, 1, M), idxe3),
                pl.BlockSpec((1, M, D), idxe3),
                pl.BlockSpec((1, M, D), idxe3),
                pl.BlockSpec((HS, 64), idx0),
                pl.BlockSpec((1, 64), idx0),
                pl.BlockSpec((64, 2), idx0),
                pl.BlockSpec((1, 2), idx0),
            ],
            out_specs=[
                pl.BlockSpec((TPE, 2 * D), idxe),
                pl.BlockSpec((1, 16), idx0),
            ],
            out_shape=[
                jax.ShapeDtypeStruct((HTOK, 2 * D), jnp.float32),
                jax.ShapeDtypeStruct((1, 16), jnp.float32),
            ],
            scratch_shapes=[pltpu.SMEM((1,), jnp.float32)],
        )(xs, ops, oat, dec_W, dec_b, psr_lut, atk_lut, w1, b1, w2, b2)

    return _call


def kernel(ctx, expert_idx, dec_W, dec_b, copy_W1, copy_b1, copy_W2, copy_b2,
           psr_lut, atk_lut, ori_psr, ori_atk):
    eidx = expert_idx.astype(jnp.int32)
    db3 = dec_b.reshape(E, 1, M)
    b1r = copy_b1.reshape(1, 64)
    b2r = copy_b2.reshape(1, 2)
    xs1, ops1, oat1 = _make_sc_gather()(ctx, ori_psr, ori_atk, eidx[:EH])
    xs2, ops2, oat2 = _make_sc_gather()(ctx, ori_psr, ori_atk, eidx[EH:])
    b1, ent1 = _make_tc_experts(0)(xs1, ops1, oat1, dec_W, db3, psr_lut,
                                   atk_lut, copy_W1, b1r, copy_W2, b2r)
    b2, ent2 = _make_tc_experts(1)(xs2, ops2, oat2, dec_W, db3, psr_lut,
                                   atk_lut, copy_W1, b1r, copy_W2, b2r)
    return _make_sc_scatter()(b1, b2, eidx, ent1, ent2)
